# Initial kernel scaffold; baseline (speedup 1.0000x reference)
import jax
import jax.numpy as jnp
from jax.experimental import pallas as pl


def kernel(x, W1, W2, rel_emb, src, rel, dst):
    raise NotImplementedError("write your pallas kernel here")


# speedup vs baseline: 22.2384x; 22.2384x over previous
import functools

import jax
import jax.numpy as jnp
import numpy as np
from jax import lax
from jax.experimental import pallas as pl
from jax.experimental.pallas import tpu as pltpu
from jax.experimental.pallas import tpu_sc as plsc

N = 10000
NREL = 10
E = 320000
NFEAT = 128
NHID = 16
OUT = 64
NP = 2 * NREL
HALF = NREL * N
DUMP = HALF
HROWS = 100096

NTILE = 16
EPAD = 327680
EPT = EPAD // NTILE
SUP = 640
NSUP = EPT // SUP
NCK = SUP // 128


def _agg_body(layer, with_counts, *refs):
    if with_counts:
        (sp, rp, dp, table, u_out, cnt_out0, cnt_out1,
         sv, rv, dv, gidx, sidx, rows, ones, zv1, u_sh, cnt_sh, gsem) = refs
    else:
        (sp, rp, dp, table, u_out,
         sv, rv, dv, gidx, sidx, rows, u_sh, gsem) = refs

    c = lax.axis_index("c")
    s = lax.axis_index("s")
    zvec = jnp.zeros((16,), jnp.float32)

    def zero_body(i, _):
        rows[i, :] = zvec
        return 0

    lax.fori_loop(0, 391, zero_body, 0)
    if with_counts:
        def zero1_body(i, _):
            zv1[pl.ds(i * 16, 16)] = zvec
            return 0

        lax.fori_loop(0, 391, zero1_body, 0)
        for k in range(8):
            ones[pl.ds(k * 16, 16)] = jnp.ones((16,), jnp.float32)

    for t in range(16):
        pltpu.sync_copy(rows.at[pl.ds(0, 391)],
                        u_sh.at[pl.ds(s * 6256 + t * 391, 391)])
    if with_counts:
        pltpu.sync_copy(zv1, cnt_sh.at[pl.ds(s * 6256, 6256)])

    plsc.subcore_barrier()

    tile_base = s * EPT
    iota16 = lax.iota(jnp.int32, 16)

    def super_body(u, _):
        base = tile_base + u * SUP
        pltpu.sync_copy(sp.at[pl.ds(base, SUP)], sv)
        pltpu.sync_copy(rp.at[pl.ds(base, SUP)], rv)
        pltpu.sync_copy(dp.at[pl.ds(base, SUP)], dv)
        for j in range(NCK):
            for k in range(8):
                off = j * 128 + k * 16
                svv = sv[pl.ds(off, 16)]
                rvv = rv[pl.ds(off, 16)]
                dvv = dv[pl.ds(off, 16)]
                is0 = c == 0
                if layer == 1:
                    gi = jnp.where(is0, svv * NP + rvv, dvv * NP + rvv + NREL)
                else:
                    gi = jnp.where(is0, svv, dvv)
                si = jnp.where(is0, dvv * NREL + rvv, svv * NREL + rvv)
                pos = base + off + iota16
                si = jnp.where(pos < E, si, DUMP)
                gidx[j, pl.ds(k * 16, 16)] = gi
                sidx[j, pl.ds(k * 16, 16)] = si
        descs = [
            pltpu.async_copy(table.at[gidx.at[j]],
                             rows.at[pl.ds(j * 128, 128)], gsem)
            for j in range(NCK)
        ]
        for d in descs:
            d.wait()
        for j in range(NCK):
            pltpu.sync_copy(rows.at[pl.ds(j * 128, 128)],
                            u_sh.at[sidx.at[j]], add=True)
            if with_counts:
                pltpu.sync_copy(ones, cnt_sh.at[sidx.at[j]], add=True)
        return 0

    lax.fori_loop(0, NSUP, super_body, 0)

    plsc.subcore_barrier()

    pltpu.sync_copy(u_sh.at[pl.ds(s * 6248, 6248)],
                    u_out.at[c, pl.ds(s * 6248, 6248)])

    @pl.when(s == 15)
    def _():
        pltpu.sync_copy(u_sh.at[pl.ds(99968, 32)],
                        u_out.at[c, pl.ds(99968, 32)])

    if with_counts:
        @pl.when(jnp.logical_and(s == 0, c == 0))
        def _():
            pltpu.sync_copy(cnt_sh.at[pl.ds(0, HALF)], cnt_out0)

        @pl.when(jnp.logical_and(s == 0, c == 1))
        def _():
            pltpu.sync_copy(cnt_sh.at[pl.ds(0, HALF)], cnt_out1)


def _make_agg(layer, with_counts, table_rows):
    mesh = plsc.VectorSubcoreMesh(core_axis_name="c", subcore_axis_name="s")
    out_type = [jax.ShapeDtypeStruct((2, HALF, NHID), jnp.float32)]
    scratch = [
        pltpu.VMEM((SUP,), jnp.int32),
        pltpu.VMEM((SUP,), jnp.int32),
        pltpu.VMEM((SUP,), jnp.int32),
        pltpu.VMEM((NCK, 128), jnp.int32),
        pltpu.VMEM((NCK, 128), jnp.int32),
        pltpu.VMEM((SUP, NHID), jnp.float32),
    ]
    if with_counts:
        out_type.append(jax.ShapeDtypeStruct((HALF,), jnp.float32))
        out_type.append(jax.ShapeDtypeStruct((HALF,), jnp.float32))
        scratch.append(pltpu.VMEM((128,), jnp.float32))
    if with_counts:
        scratch.append(pltpu.VMEM((6256,), jnp.float32))
    scratch.append(pltpu.VMEM_SHARED((HROWS, NHID), jnp.float32))
    if with_counts:
        scratch.append(pltpu.VMEM_SHARED((HROWS,), jnp.float32))
    scratch.append(pltpu.SemaphoreType.DMA)
    return pl.kernel(
        functools.partial(_agg_body, layer, with_counts),
        out_type=out_type,
        mesh=mesh,
        scratch_types=scratch,
        compiler_params=pltpu.CompilerParams(use_tc_tiling_on_sc=False),
    )


def _xw1_body(x_ref, w_ref, out_ref):
    out_ref[...] = jnp.dot(x_ref[...], w_ref[...],
                           preferred_element_type=jnp.float32)


def _xw1(x, W1cat):
    return pl.pallas_call(
        _xw1_body,
        out_shape=jax.ShapeDtypeStruct((N, NP * NHID), jnp.float32),
    )(x, W1cat)


_KRON = np.kron(np.eye(NREL, dtype=np.float32), np.ones((1, NHID), np.float32))
_SUMT = np.tile(np.eye(NHID, dtype=np.float32), (NREL, 1))


def _h1_body(u0_ref, u1_ref, c0_ref, c1_ref, x_ref, w_ref, kron_ref, sumt_ref,
             out_ref):
    invb0 = jnp.dot(1.0 / jnp.maximum(c0_ref[...], 1.0), kron_ref[...],
                    preferred_element_type=jnp.float32)
    invb1 = jnp.dot(1.0 / jnp.maximum(c1_ref[...], 1.0), kron_ref[...],
                    preferred_element_type=jnp.float32)
    s = jnp.dot(u0_ref[...] * invb0 + u1_ref[...] * invb1, sumt_ref[...],
                preferred_element_type=jnp.float32)
    s = s + jnp.dot(x_ref[...], w_ref[...], preferred_element_type=jnp.float32)
    out_ref[...] = jnp.maximum(s, 0.0)


_NB = 2000


def _h1(u0, u1, c0, c1, x, W1self):
    return pl.pallas_call(
        _h1_body,
        grid=(N // _NB,),
        in_specs=[
            pl.BlockSpec((_NB, NREL * NHID), lambda i: (i, 0)),
            pl.BlockSpec((_NB, NREL * NHID), lambda i: (i, 0)),
            pl.BlockSpec((_NB, NREL), lambda i: (i, 0)),
            pl.BlockSpec((_NB, NREL), lambda i: (i, 0)),
            pl.BlockSpec((_NB, NFEAT), lambda i: (i, 0)),
            pl.BlockSpec((NFEAT, NHID), lambda i: (0, 0)),
            pl.BlockSpec((NREL, NREL * NHID), lambda i: (0, 0)),
            pl.BlockSpec((NREL * NHID, NHID), lambda i: (0, 0)),
        ],
        out_specs=pl.BlockSpec((_NB, NHID), lambda i: (i, 0)),
        out_shape=jax.ShapeDtypeStruct((N, NHID), jnp.float32),
    )(u0, u1, c0, c1, x, W1self, jnp.asarray(_KRON), jnp.asarray(_SUMT))


_TILE64 = np.tile(np.eye(OUT, dtype=np.float32), (1, NREL))


def _h2_body(u0_ref, u1_ref, c0_ref, c1_ref, h1_ref, w0_ref, w1_ref, ws_ref,
             kron_ref, til_ref, emb_ref, out_ref, q_ref):
    invb0 = jnp.dot(1.0 / jnp.maximum(c0_ref[...], 1.0), kron_ref[...],
                    preferred_element_type=jnp.float32)
    invb1 = jnp.dot(1.0 / jnp.maximum(c1_ref[...], 1.0), kron_ref[...],
                    preferred_element_type=jnp.float32)
    acc = jnp.dot(h1_ref[...], ws_ref[...], preferred_element_type=jnp.float32)
    acc = acc + jnp.dot(u0_ref[...] * invb0, w0_ref[...],
                        preferred_element_type=jnp.float32)
    acc = acc + jnp.dot(u1_ref[...] * invb1, w1_ref[...],
                        preferred_element_type=jnp.float32)
    out_ref[...] = acc
    q_ref[...] = jnp.dot(acc, til_ref[...],
                         preferred_element_type=jnp.float32) * emb_ref[...]


def _h2(u0, u1, c0, c1, h1, W2a, W2b, W2self, rel_emb):
    embrow = rel_emb.reshape(1, NREL * OUT)
    return pl.pallas_call(
        _h2_body,
        grid=(N // _NB,),
        in_specs=[
            pl.BlockSpec((_NB, NREL * NHID), lambda i: (i, 0)),
            pl.BlockSpec((_NB, NREL * NHID), lambda i: (i, 0)),
            pl.BlockSpec((_NB, NREL), lambda i: (i, 0)),
            pl.BlockSpec((_NB, NREL), lambda i: (i, 0)),
            pl.BlockSpec((_NB, NHID), lambda i: (i, 0)),
            pl.BlockSpec((NREL * NHID, OUT), lambda i: (0, 0)),
            pl.BlockSpec((NREL * NHID, OUT), lambda i: (0, 0)),
            pl.BlockSpec((NHID, OUT), lambda i: (0, 0)),
            pl.BlockSpec((NREL, NREL * NHID), lambda i: (0, 0)),
            pl.BlockSpec((OUT, NREL * OUT), lambda i: (0, 0)),
            pl.BlockSpec((1, NREL * OUT), lambda i: (0, 0)),
        ],
        out_specs=[
            pl.BlockSpec((_NB, OUT), lambda i: (i, 0)),
            pl.BlockSpec((_NB, NREL * OUT), lambda i: (i, 0)),
        ],
        out_shape=[
            jax.ShapeDtypeStruct((N, OUT), jnp.float32),
            jax.ShapeDtypeStruct((N, NREL * OUT), jnp.float32),
        ],
    )(u0, u1, c0, c1, h1, W2a, W2b, W2self, jnp.asarray(_KRON),
      jnp.asarray(_TILE64), embrow)


DSUP = 640
DCK = DSUP // 128
EPW = EPAD // 32
NDSUP = EPW // DSUP


def _dec_gather_body(sp, rp, dp, qt, ht, g_out,
                     sv, rv, dv, qidx, hidx, qbuf, hbuf, gsem):
    c = lax.axis_index("c")
    s = lax.axis_index("s")
    wid = s * 2 + c
    base0 = wid * EPW
    iota16 = lax.iota(jnp.int32, 16)

    def super_body(u, _):
        base = base0 + u * DSUP
        pltpu.sync_copy(sp.at[pl.ds(base, DSUP)], sv)
        pltpu.sync_copy(rp.at[pl.ds(base, DSUP)], rv)
        pltpu.sync_copy(dp.at[pl.ds(base, DSUP)], dv)
        for j in range(DCK):
            for k in range(8):
                off = j * 128 + k * 16
                svv = sv[pl.ds(off, 16)]
                rvv = rv[pl.ds(off, 16)]
                dvv = dv[pl.ds(off, 16)]
                qidx[j, pl.ds(k * 16, 16)] = svv * NREL + rvv
                hidx[j, pl.ds(k * 16, 16)] = dvv
        descs = [
            pltpu.async_copy(qt.at[qidx.at[j]],
                             qbuf.at[pl.ds(j * 128, 128)], gsem)
            for j in range(DCK)
        ] + [
            pltpu.async_copy(ht.at[hidx.at[j]],
                             hbuf.at[pl.ds(j * 128, 128)], gsem)
            for j in range(DCK)
        ]
        for d in descs:
            d.wait()
        pltpu.sync_copy(qbuf, g_out.at[pl.ds(base, DSUP), pl.ds(0, OUT)])
        pltpu.sync_copy(hbuf, g_out.at[pl.ds(base, DSUP), pl.ds(OUT, OUT)])
        return 0

    lax.fori_loop(0, NDSUP, super_body, 0)


def _dec_gather(sp, rp, dp, qtab, h2):
    mesh = plsc.VectorSubcoreMesh(core_axis_name="c", subcore_axis_name="s")
    return pl.kernel(
        _dec_gather_body,
        out_type=jax.ShapeDtypeStruct((EPAD, 2 * OUT), jnp.float32),
        mesh=mesh,
        scratch_types=[
            pltpu.VMEM((DSUP,), jnp.int32),
            pltpu.VMEM((DSUP,), jnp.int32),
            pltpu.VMEM((DSUP,), jnp.int32),
            pltpu.VMEM((DCK, 128), jnp.int32),
            pltpu.VMEM((DCK, 128), jnp.int32),
            pltpu.VMEM((DSUP, OUT), jnp.float32),
            pltpu.VMEM((DSUP, OUT), jnp.float32),
            pltpu.SemaphoreType.DMA,
        ],
        compiler_params=pltpu.CompilerParams(use_tc_tiling_on_sc=False),
    )(sp, rp, dp, qtab, h2)


_DB = 12800


def _dec_body(g_ref, o_ref):
    o_ref[0] = jnp.sum(g_ref[:, :OUT] * g_ref[:, OUT:], axis=1).reshape(
        8, _DB // 8)


def _decode(g):
    out = pl.pallas_call(
        _dec_body,
        grid=(E // _DB,),
        in_specs=[pl.BlockSpec((_DB, 2 * OUT), lambda i: (i, 0))],
        out_specs=pl.BlockSpec((1, 8, _DB // 8), lambda i: (i, 0, 0)),
        out_shape=jax.ShapeDtypeStruct((E // _DB, 8, _DB // 8), jnp.float32),
    )(g)
    return out.reshape(E)


def kernel(x, W1, W2, rel_emb, src, rel, dst):
    src = src.astype(jnp.int32)
    rel = rel.astype(jnp.int32)
    dst = dst.astype(jnp.int32)

    pad = jnp.zeros((EPAD - E,), jnp.int32)
    sp = jnp.concatenate([src, pad])
    rp = jnp.concatenate([rel, pad])
    dp = jnp.concatenate([dst, pad])

    W1cat = W1[:NP].transpose(1, 0, 2).reshape(NFEAT, NP * NHID)
    XW = _xw1(x, W1cat)
    XWf = XW.reshape(N * NP, NHID)

    U1, cnt0, cnt1 = _make_agg(1, True, NP * N)(sp, rp, dp, XWf)
    u10 = U1[0].reshape(N, NREL * NHID)
    u11 = U1[1].reshape(N, NREL * NHID)
    c0 = cnt0.reshape(N, NREL)
    c1 = cnt1.reshape(N, NREL)

    h1 = _h1(u10, u11, c0, c1, x, W1[NP])

    (U2,) = _make_agg(2, False, N)(sp, rp, dp, h1)
    u20 = U2[0].reshape(N, NREL * NHID)
    u21 = U2[1].reshape(N, NREL * NHID)

    h2, qtab = _h2(u20, u21, c0, c1, h1,
                   W2[:NREL].reshape(NREL * NHID, OUT),
                   W2[NREL:NP].reshape(NREL * NHID, OUT), W2[NP], rel_emb)

    g = _dec_gather(sp, rp, dp, qtab.reshape(N * NREL, OUT), h2)
    return _decode(g)


# speedup vs baseline: 22.8849x; 1.0291x over previous
import functools

import jax
import jax.numpy as jnp
import numpy as np
from jax import lax
from jax.experimental import pallas as pl
from jax.experimental.pallas import tpu as pltpu
from jax.experimental.pallas import tpu_sc as plsc

N = 10000
NREL = 10
E = 320000
NFEAT = 128
NHID = 16
OUT = 64
NP = 2 * NREL
HALF = NREL * N
DUMP = HALF
HROWS = 100096

NTILE = 16
EPAD = 327680
EPT = EPAD // NTILE
SUP = 640
NSUP = EPT // SUP
NCK = SUP // 128


def _agg_body(layer, with_counts, *refs):
    if with_counts:
        (sp, rp, dp, table, u_out, cnt_out0, cnt_out1,
         sv, rv, dv, gidx, sidx, rows, ones, zv1, u_sh, cnt_sh, gsem) = refs
    else:
        (sp, rp, dp, table, u_out,
         sv, rv, dv, gidx, sidx, rows, u_sh, gsem) = refs

    c = lax.axis_index("c")
    s = lax.axis_index("s")
    zvec = jnp.zeros((16,), jnp.float32)

    def zero_body(i, _):
        rows[i, :] = zvec
        return 0

    lax.fori_loop(0, 391, zero_body, 0)
    if with_counts:
        def zero1_body(i, _):
            zv1[pl.ds(i * 16, 16)] = zvec
            return 0

        lax.fori_loop(0, 391, zero1_body, 0)
        for k in range(8):
            ones[pl.ds(k * 16, 16)] = jnp.ones((16,), jnp.float32)

    for t in range(16):
        pltpu.sync_copy(rows.at[pl.ds(0, 391)],
                        u_sh.at[pl.ds(s * 6256 + t * 391, 391)])
    if with_counts:
        pltpu.sync_copy(zv1, cnt_sh.at[pl.ds(s * 6256, 6256)])

    plsc.subcore_barrier()

    tile_base = s * EPT
    iota16 = lax.iota(jnp.int32, 16)

    def super_body(u, _):
        base = tile_base + u * SUP
        pltpu.sync_copy(sp.at[pl.ds(base, SUP)], sv)
        pltpu.sync_copy(rp.at[pl.ds(base, SUP)], rv)
        pltpu.sync_copy(dp.at[pl.ds(base, SUP)], dv)
        for j in range(NCK):
            for k in range(8):
                off = j * 128 + k * 16
                svv = sv[pl.ds(off, 16)]
                rvv = rv[pl.ds(off, 16)]
                dvv = dv[pl.ds(off, 16)]
                is0 = c == 0
                if layer == 1:
                    gi = jnp.where(is0, svv * NP + rvv, dvv * NP + rvv + NREL)
                else:
                    gi = jnp.where(is0, svv, dvv)
                si = jnp.where(is0, dvv * NREL + rvv, svv * NREL + rvv)
                pos = base + off + iota16
                si = jnp.where(pos < E, si, DUMP)
                gidx[j, pl.ds(k * 16, 16)] = gi
                sidx[j, pl.ds(k * 16, 16)] = si
        descs = [
            pltpu.async_copy(table.at[gidx.at[j]],
                             rows.at[pl.ds(j * 128, 128)], gsem)
            for j in range(NCK)
        ]
        for d in descs:
            d.wait()
        for j in range(NCK):
            pltpu.sync_copy(rows.at[pl.ds(j * 128, 128)],
                            u_sh.at[sidx.at[j]], add=True)
            if with_counts:
                pltpu.sync_copy(ones, cnt_sh.at[sidx.at[j]], add=True)
        return 0

    lax.fori_loop(0, NSUP, super_body, 0)

    plsc.subcore_barrier()

    pltpu.sync_copy(u_sh.at[pl.ds(s * 6248, 6248)],
                    u_out.at[c, pl.ds(s * 6248, 6248)])

    @pl.when(s == 15)
    def _():
        pltpu.sync_copy(u_sh.at[pl.ds(99968, 32)],
                        u_out.at[c, pl.ds(99968, 32)])

    if with_counts:
        @pl.when(jnp.logical_and(s == 0, c == 0))
        def _():
            pltpu.sync_copy(cnt_sh.at[pl.ds(0, HALF)], cnt_out0)

        @pl.when(jnp.logical_and(s == 0, c == 1))
        def _():
            pltpu.sync_copy(cnt_sh.at[pl.ds(0, HALF)], cnt_out1)


def _make_agg(layer, with_counts, table_rows):
    mesh = plsc.VectorSubcoreMesh(core_axis_name="c", subcore_axis_name="s")
    out_type = [jax.ShapeDtypeStruct((2, HALF, NHID), jnp.float32)]
    scratch = [
        pltpu.VMEM((SUP,), jnp.int32),
        pltpu.VMEM((SUP,), jnp.int32),
        pltpu.VMEM((SUP,), jnp.int32),
        pltpu.VMEM((NCK, 128), jnp.int32),
        pltpu.VMEM((NCK, 128), jnp.int32),
        pltpu.VMEM((SUP, NHID), jnp.float32),
    ]
    if with_counts:
        out_type.append(jax.ShapeDtypeStruct((HALF,), jnp.float32))
        out_type.append(jax.ShapeDtypeStruct((HALF,), jnp.float32))
        scratch.append(pltpu.VMEM((128,), jnp.float32))
    if with_counts:
        scratch.append(pltpu.VMEM((6256,), jnp.float32))
    scratch.append(pltpu.VMEM_SHARED((HROWS, NHID), jnp.float32))
    if with_counts:
        scratch.append(pltpu.VMEM_SHARED((HROWS,), jnp.float32))
    scratch.append(pltpu.SemaphoreType.DMA)
    return pl.kernel(
        functools.partial(_agg_body, layer, with_counts),
        out_type=out_type,
        mesh=mesh,
        scratch_types=scratch,
        compiler_params=pltpu.CompilerParams(use_tc_tiling_on_sc=False),
    )


def _xw1_body(x_ref, w_ref, out_ref):
    out_ref[...] = jnp.dot(x_ref[...], w_ref[...],
                           preferred_element_type=jnp.float32)


def _xw1(x, W1cat):
    return pl.pallas_call(
        _xw1_body,
        out_shape=jax.ShapeDtypeStruct((N, NP * NHID), jnp.float32),
    )(x, W1cat)


_KRON = np.kron(np.eye(NREL, dtype=np.float32), np.ones((1, NHID), np.float32))
_SUMT = np.tile(np.eye(NHID, dtype=np.float32), (NREL, 1))


def _h1_body(u0_ref, u1_ref, c0_ref, c1_ref, x_ref, w_ref, kron_ref, sumt_ref,
             out_ref):
    invb0 = jnp.dot(1.0 / jnp.maximum(c0_ref[...], 1.0), kron_ref[...],
                    preferred_element_type=jnp.float32)
    invb1 = jnp.dot(1.0 / jnp.maximum(c1_ref[...], 1.0), kron_ref[...],
                    preferred_element_type=jnp.float32)
    s = jnp.dot(u0_ref[...] * invb0 + u1_ref[...] * invb1, sumt_ref[...],
                preferred_element_type=jnp.float32)
    s = s + jnp.dot(x_ref[...], w_ref[...], preferred_element_type=jnp.float32)
    out_ref[...] = jnp.maximum(s, 0.0)


_NB = 2000


def _h1(u0, u1, c0, c1, x, W1self):
    return pl.pallas_call(
        _h1_body,
        grid=(N // _NB,),
        in_specs=[
            pl.BlockSpec((_NB, NREL * NHID), lambda i: (i, 0)),
            pl.BlockSpec((_NB, NREL * NHID), lambda i: (i, 0)),
            pl.BlockSpec((_NB, NREL), lambda i: (i, 0)),
            pl.BlockSpec((_NB, NREL), lambda i: (i, 0)),
            pl.BlockSpec((_NB, NFEAT), lambda i: (i, 0)),
            pl.BlockSpec((NFEAT, NHID), lambda i: (0, 0)),
            pl.BlockSpec((NREL, NREL * NHID), lambda i: (0, 0)),
            pl.BlockSpec((NREL * NHID, NHID), lambda i: (0, 0)),
        ],
        out_specs=pl.BlockSpec((_NB, NHID), lambda i: (i, 0)),
        out_shape=jax.ShapeDtypeStruct((N, NHID), jnp.float32),
    )(u0, u1, c0, c1, x, W1self, jnp.asarray(_KRON), jnp.asarray(_SUMT))


_TILE64 = np.tile(np.eye(OUT, dtype=np.float32), (1, NREL))


def _h2_body(u0_ref, u1_ref, c0_ref, c1_ref, h1_ref, w0_ref, w1_ref, ws_ref,
             kron_ref, til_ref, emb_ref, out_ref, q_ref):
    invb0 = jnp.dot(1.0 / jnp.maximum(c0_ref[...], 1.0), kron_ref[...],
                    preferred_element_type=jnp.float32)
    invb1 = jnp.dot(1.0 / jnp.maximum(c1_ref[...], 1.0), kron_ref[...],
                    preferred_element_type=jnp.float32)
    acc = jnp.dot(h1_ref[...], ws_ref[...], preferred_element_type=jnp.float32)
    acc = acc + jnp.dot(u0_ref[...] * invb0, w0_ref[...],
                        preferred_element_type=jnp.float32)
    acc = acc + jnp.dot(u1_ref[...] * invb1, w1_ref[...],
                        preferred_element_type=jnp.float32)
    out_ref[...] = acc
    q_ref[...] = jnp.dot(acc, til_ref[...],
                         preferred_element_type=jnp.float32) * emb_ref[...]


def _h2(u0, u1, c0, c1, h1, W2a, W2b, W2self, rel_emb):
    embrow = rel_emb.reshape(1, NREL * OUT)
    return pl.pallas_call(
        _h2_body,
        grid=(N // _NB,),
        in_specs=[
            pl.BlockSpec((_NB, NREL * NHID), lambda i: (i, 0)),
            pl.BlockSpec((_NB, NREL * NHID), lambda i: (i, 0)),
            pl.BlockSpec((_NB, NREL), lambda i: (i, 0)),
            pl.BlockSpec((_NB, NREL), lambda i: (i, 0)),
            pl.BlockSpec((_NB, NHID), lambda i: (i, 0)),
            pl.BlockSpec((NREL * NHID, OUT), lambda i: (0, 0)),
            pl.BlockSpec((NREL * NHID, OUT), lambda i: (0, 0)),
            pl.BlockSpec((NHID, OUT), lambda i: (0, 0)),
            pl.BlockSpec((NREL, NREL * NHID), lambda i: (0, 0)),
            pl.BlockSpec((OUT, NREL * OUT), lambda i: (0, 0)),
            pl.BlockSpec((1, NREL * OUT), lambda i: (0, 0)),
        ],
        out_specs=[
            pl.BlockSpec((_NB, OUT), lambda i: (i, 0)),
            pl.BlockSpec((_NB, NREL * OUT), lambda i: (i, 0)),
        ],
        out_shape=[
            jax.ShapeDtypeStruct((N, OUT), jnp.float32),
            jax.ShapeDtypeStruct((N, NREL * OUT), jnp.float32),
        ],
    )(u0, u1, c0, c1, h1, W2a, W2b, W2self, jnp.asarray(_KRON),
      jnp.asarray(_TILE64), embrow)


DSUP = 640
DCK = DSUP // 128
EPW = EPAD // 32
NDSUP = EPW // DSUP


def _dec_gather_body(sp, rp, dp, qt, ht, gp_out,
                     sv, rv, dv, qidx, hidx, qbuf, hbuf, pbuf, gsem):
    c = lax.axis_index("c")
    s = lax.axis_index("s")
    wid = s * 2 + c
    base0 = wid * EPW

    def super_body(u, _):
        base = base0 + u * DSUP
        pltpu.sync_copy(sp.at[pl.ds(base, DSUP)], sv)
        pltpu.sync_copy(rp.at[pl.ds(base, DSUP)], rv)
        pltpu.sync_copy(dp.at[pl.ds(base, DSUP)], dv)
        for j in range(DCK):
            for k in range(8):
                off = j * 128 + k * 16
                svv = sv[pl.ds(off, 16)]
                rvv = rv[pl.ds(off, 16)]
                dvv = dv[pl.ds(off, 16)]
                qidx[j, pl.ds(k * 16, 16)] = svv * NREL + rvv
                hidx[j, pl.ds(k * 16, 16)] = dvv
        descs = [
            pltpu.async_copy(qt.at[qidx.at[j]],
                             qbuf.at[pl.ds(j * 128, 128)], gsem)
            for j in range(DCK)
        ] + [
            pltpu.async_copy(ht.at[hidx.at[j]],
                             hbuf.at[pl.ds(j * 128, 128)], gsem)
            for j in range(DCK)
        ]
        for d in descs:
            d.wait()

        def edge_body(e, _):
            acc = ((qbuf[e, pl.ds(0, 16)] * hbuf[e, pl.ds(0, 16)]
                    + qbuf[e, pl.ds(16, 16)] * hbuf[e, pl.ds(16, 16)])
                   + (qbuf[e, pl.ds(32, 16)] * hbuf[e, pl.ds(32, 16)]
                      + qbuf[e, pl.ds(48, 16)] * hbuf[e, pl.ds(48, 16)]))
            pbuf[pl.ds(e * 16, 16)] = acc
            return 0

        lax.fori_loop(0, DSUP, edge_body, 0)
        pltpu.sync_copy(pbuf, gp_out.at[pl.ds(base * 16, DSUP * 16)])
        return 0

    lax.fori_loop(0, NDSUP, super_body, 0)


def _dec_gather(sp, rp, dp, qtab, h2):
    mesh = plsc.VectorSubcoreMesh(core_axis_name="c", subcore_axis_name="s")
    return pl.kernel(
        _dec_gather_body,
        out_type=jax.ShapeDtypeStruct((EPAD * 16,), jnp.float32),
        mesh=mesh,
        scratch_types=[
            pltpu.VMEM((DSUP,), jnp.int32),
            pltpu.VMEM((DSUP,), jnp.int32),
            pltpu.VMEM((DSUP,), jnp.int32),
            pltpu.VMEM((DCK, 128), jnp.int32),
            pltpu.VMEM((DCK, 128), jnp.int32),
            pltpu.VMEM((DSUP, OUT), jnp.float32),
            pltpu.VMEM((DSUP, OUT), jnp.float32),
            pltpu.VMEM((DSUP * 16,), jnp.float32),
            pltpu.SemaphoreType.DMA,
        ],
        compiler_params=pltpu.CompilerParams(use_tc_tiling_on_sc=False),
    )(sp, rp, dp, qtab, h2)


_SUM16 = np.kron(np.eye(8, dtype=np.float32), np.ones((16, 1), np.float32))
_GROWS = EPAD * 16 // 128
_GB = 4096


def _dec_body(g_ref, k_ref, o_ref):
    o_ref[...] = jnp.dot(g_ref[...], k_ref[...],
                         preferred_element_type=jnp.float32)


def _decode(gp):
    out = pl.pallas_call(
        _dec_body,
        grid=(_GROWS // _GB,),
        in_specs=[
            pl.BlockSpec((_GB, 128), lambda i: (i, 0)),
            pl.BlockSpec((128, 8), lambda i: (0, 0)),
        ],
        out_specs=pl.BlockSpec((_GB, 8), lambda i: (i, 0)),
        out_shape=jax.ShapeDtypeStruct((_GROWS, 8), jnp.float32),
    )(gp.reshape(_GROWS, 128), jnp.asarray(_SUM16))
    return out.reshape(EPAD)[:E]


def kernel(x, W1, W2, rel_emb, src, rel, dst):
    src = src.astype(jnp.int32)
    rel = rel.astype(jnp.int32)
    dst = dst.astype(jnp.int32)

    pad = jnp.zeros((EPAD - E,), jnp.int32)
    sp = jnp.concatenate([src, pad])
    rp = jnp.concatenate([rel, pad])
    dp = jnp.concatenate([dst, pad])

    W1cat = W1[:NP].transpose(1, 0, 2).reshape(NFEAT, NP * NHID)
    XW = _xw1(x, W1cat)
    XWf = XW.reshape(N * NP, NHID)

    U1, cnt0, cnt1 = _make_agg(1, True, NP * N)(sp, rp, dp, XWf)
    u10 = U1[0].reshape(N, NREL * NHID)
    u11 = U1[1].reshape(N, NREL * NHID)
    c0 = cnt0.reshape(N, NREL)
    c1 = cnt1.reshape(N, NREL)

    h1 = _h1(u10, u11, c0, c1, x, W1[NP])

    (U2,) = _make_agg(2, False, N)(sp, rp, dp, h1)
    u20 = U2[0].reshape(N, NREL * NHID)
    u21 = U2[1].reshape(N, NREL * NHID)

    h2, qtab = _h2(u20, u21, c0, c1, h1,
                   W2[:NREL].reshape(NREL * NHID, OUT),
                   W2[NREL:NP].reshape(NREL * NHID, OUT), W2[NP], rel_emb)

    gp = _dec_gather(sp, rp, dp, qtab.reshape(N * NREL, OUT), h2)
    return _decode(gp)


# speedup vs baseline: 28.9888x; 1.2667x over previous
import functools

import jax
import jax.numpy as jnp
import numpy as np
from jax import lax
from jax.experimental import pallas as pl
from jax.experimental.pallas import tpu as pltpu
from jax.experimental.pallas import tpu_sc as plsc

N = 10000
NREL = 10
E = 320000
NFEAT = 128
NHID = 16
OUT = 64
NP = 2 * NREL
NPAD = 10240
HALF = NREL * NPAD
DUMP = HALF
HROWS = HALF + 64
UCROWS = HALF * NHID // 128

NTILE = 16
EPAD = 327680
EPT = EPAD // NTILE
SUP = 640
NSUP = EPT // SUP
NCK = SUP // 128


def _agg_body(layer, with_counts, *refs):
    if with_counts:
        (sp, rp, dp, table, u_out, cnt_out0, cnt_out1,
         sv, rv, dv, gidx, sidx, rows, r128, ones, zv1, u_sh, cnt_sh,
         gsem) = refs
    else:
        (sp, rp, dp, table, u_out,
         sv, rv, dv, gidx, sidx, rows, r128, u_sh, gsem) = refs

    c = lax.axis_index("c")
    s = lax.axis_index("s")
    zvec = jnp.zeros((16,), jnp.float32)

    def zero_body(i, _):
        rows[i, :] = zvec
        return 0

    lax.fori_loop(0, SUP, zero_body, 0)
    if with_counts:
        def zero1_body(i, _):
            zv1[pl.ds(i * 16, 16)] = zvec
            return 0

        lax.fori_loop(0, 100, zero1_body, 0)
        for k in range(8):
            ones[pl.ds(k * 16, 16)] = jnp.ones((16,), jnp.float32)

    for t in range(10):
        pltpu.sync_copy(rows, u_sh.at[pl.ds(s * 6404 + t * 640, 640)])
    pltpu.sync_copy(rows.at[pl.ds(0, 4)], u_sh.at[pl.ds(s * 6404 + 6400, 4)])
    if with_counts:
        for t in range(4):
            pltpu.sync_copy(zv1, cnt_sh.at[pl.ds(s * 6400 + t * 1600, 1600)])

        @pl.when(s == 0)
        def _():
            pltpu.sync_copy(zv1.at[pl.ds(0, 64)], cnt_sh.at[pl.ds(HALF, 64)])

    plsc.subcore_barrier()

    tile_base = s * EPT
    iota16 = lax.iota(jnp.int32, 16)

    def super_body(u, _):
        base = tile_base + u * SUP
        pltpu.sync_copy(sp.at[pl.ds(base, SUP)], sv)
        pltpu.sync_copy(rp.at[pl.ds(base, SUP)], rv)
        pltpu.sync_copy(dp.at[pl.ds(base, SUP)], dv)
        for j in range(NCK):
            for k in range(8):
                off = j * 128 + k * 16
                svv = sv[pl.ds(off, 16)]
                rvv = rv[pl.ds(off, 16)]
                dvv = dv[pl.ds(off, 16)]
                is0 = c == 0
                if layer == 1:
                    gi = jnp.where(is0, svv * NP + rvv, dvv * NP + rvv + NREL)
                else:
                    gi = jnp.where(is0, svv, dvv)
                si = jnp.where(is0, dvv * NREL + rvv, svv * NREL + rvv)
                pos = base + off + iota16
                si = jnp.where(pos < E, si, DUMP)
                gidx[j, pl.ds(k * 16, 16)] = gi
                sidx[j, pl.ds(k * 16, 16)] = si
        descs = [
            pltpu.async_copy(table.at[gidx.at[j]],
                             rows.at[pl.ds(j * 128, 128)], gsem)
            for j in range(NCK)
        ]
        for d in descs:
            d.wait()
        for j in range(NCK):
            pltpu.sync_copy(rows.at[pl.ds(j * 128, 128)],
                            u_sh.at[sidx.at[j]], add=True)
            if with_counts:
                pltpu.sync_copy(ones, cnt_sh.at[sidx.at[j]], add=True)
        return 0

    lax.fori_loop(0, NSUP, super_body, 0)

    plsc.subcore_barrier()

    def repack_body(i8, _):
        for k in range(8):
            r128[i8, pl.ds(k * 16, 16)] = rows[i8 * 8 + k, :]
        return 0

    for t in range(20):
        pltpu.sync_copy(u_sh.at[pl.ds(s * 6400 + t * 320, 320)],
                        rows.at[pl.ds(0, 320)])
        lax.fori_loop(0, 40, repack_body, 0)
        pltpu.sync_copy(r128,
                        u_out.at[pl.ds(c * UCROWS + s * 800 + t * 40, 40)])

    if with_counts:
        @pl.when(jnp.logical_and(s == 0, c == 0))
        def _():
            pltpu.sync_copy(cnt_sh.at[pl.ds(0, HALF)], cnt_out0)

        @pl.when(jnp.logical_and(s == 0, c == 1))
        def _():
            pltpu.sync_copy(cnt_sh.at[pl.ds(0, HALF)], cnt_out1)


def _make_agg(layer, with_counts, table_rows):
    mesh = plsc.VectorSubcoreMesh(core_axis_name="c", subcore_axis_name="s")
    out_type = [jax.ShapeDtypeStruct((2 * UCROWS, 128), jnp.float32)]
    scratch = [
        pltpu.VMEM((SUP,), jnp.int32),
        pltpu.VMEM((SUP,), jnp.int32),
        pltpu.VMEM((SUP,), jnp.int32),
        pltpu.VMEM((NCK, 128), jnp.int32),
        pltpu.VMEM((NCK, 128), jnp.int32),
        pltpu.VMEM((SUP, NHID), jnp.float32),
        pltpu.VMEM((40, 128), jnp.float32),
    ]
    if with_counts:
        out_type.append(jax.ShapeDtypeStruct((HALF,), jnp.float32))
        out_type.append(jax.ShapeDtypeStruct((HALF,), jnp.float32))
        scratch.append(pltpu.VMEM((128,), jnp.float32))
    if with_counts:
        scratch.append(pltpu.VMEM((1600,), jnp.float32))
    scratch.append(pltpu.VMEM_SHARED((HROWS, NHID), jnp.float32))
    if with_counts:
        scratch.append(pltpu.VMEM_SHARED((HROWS,), jnp.float32))
    scratch.append(pltpu.SemaphoreType.DMA)
    return pl.kernel(
        functools.partial(_agg_body, layer, with_counts),
        out_type=out_type,
        mesh=mesh,
        scratch_types=scratch,
        compiler_params=pltpu.CompilerParams(use_tc_tiling_on_sc=False),
    )


def _xw1_body(x_ref, w_ref, out_ref):
    out_ref[...] = jnp.dot(x_ref[...], w_ref[...],
                           preferred_element_type=jnp.float32)


def _xw1(x, W1cat):
    return pl.pallas_call(
        _xw1_body,
        out_shape=jax.ShapeDtypeStruct((N, NP * NHID), jnp.float32),
    )(x, W1cat)


_KRON = np.kron(np.eye(NREL, dtype=np.float32), np.ones((1, NHID), np.float32))
_SUMT = np.tile(np.eye(NHID, dtype=np.float32), (NREL, 1))


_NB = 2048
_UB = _NB * NREL * NHID // 128


def _h1_body(u0_ref, u1_ref, c0_ref, c1_ref, x_ref, w_ref, kron_ref, sumt_ref,
             out_ref):
    u0 = u0_ref[...]
    u1 = u1_ref[...]
    invb0 = jnp.dot(1.0 / jnp.maximum(c0_ref[...], 1.0), kron_ref[...],
                    preferred_element_type=jnp.float32)
    invb1 = jnp.dot(1.0 / jnp.maximum(c1_ref[...], 1.0), kron_ref[...],
                    preferred_element_type=jnp.float32)
    s = jnp.dot(u0 * invb0 + u1 * invb1, sumt_ref[...],
                preferred_element_type=jnp.float32)
    s = s + jnp.dot(x_ref[...], w_ref[...], preferred_element_type=jnp.float32)
    out_ref[...] = jnp.maximum(s, 0.0)


def _h1(U, c0, c1, x, W1self):
    return pl.pallas_call(
        _h1_body,
        grid=(N // _NB,),
        in_specs=[
            pl.BlockSpec((_NB, NREL * NHID), lambda i: (i, 0)),
            pl.BlockSpec((_NB, NREL * NHID), lambda i: (i + 5, 0)),
            pl.BlockSpec((_NB, NREL), lambda i: (i, 0)),
            pl.BlockSpec((_NB, NREL), lambda i: (i, 0)),
            pl.BlockSpec((_NB, NFEAT), lambda i: (i, 0)),
            pl.BlockSpec((NFEAT, NHID), lambda i: (0, 0)),
            pl.BlockSpec((NREL, NREL * NHID), lambda i: (0, 0)),
            pl.BlockSpec((NREL * NHID, NHID), lambda i: (0, 0)),
        ],
        out_specs=pl.BlockSpec((_NB, NHID), lambda i: (i, 0)),
        out_shape=jax.ShapeDtypeStruct((N, NHID), jnp.float32),
    )(U, U, c0, c1, x, W1self, jnp.asarray(_KRON), jnp.asarray(_SUMT))


_TILE64 = np.tile(np.eye(OUT, dtype=np.float32), (1, NREL))


def _h2_body(u0_ref, u1_ref, c0_ref, c1_ref, h1_ref, w0_ref, w1_ref, ws_ref,
             kron_ref, til_ref, emb_ref, out_ref, q_ref):
    u0 = u0_ref[...]
    u1 = u1_ref[...]
    invb0 = jnp.dot(1.0 / jnp.maximum(c0_ref[...], 1.0), kron_ref[...],
                    preferred_element_type=jnp.float32)
    invb1 = jnp.dot(1.0 / jnp.maximum(c1_ref[...], 1.0), kron_ref[...],
                    preferred_element_type=jnp.float32)
    acc = jnp.dot(h1_ref[...], ws_ref[...], preferred_element_type=jnp.float32)
    acc = acc + jnp.dot(u0 * invb0, w0_ref[...],
                        preferred_element_type=jnp.float32)
    acc = acc + jnp.dot(u1 * invb1, w1_ref[...],
                        preferred_element_type=jnp.float32)
    out_ref[...] = acc
    q_ref[...] = jnp.dot(acc, til_ref[...],
                         preferred_element_type=jnp.float32) * emb_ref[...]


def _h2(U, c0, c1, h1, W2a, W2b, W2self, rel_emb):
    embrow = rel_emb.reshape(1, NREL * OUT)
    return pl.pallas_call(
        _h2_body,
        grid=(N // _NB,),
        in_specs=[
            pl.BlockSpec((_NB, NREL * NHID), lambda i: (i, 0)),
            pl.BlockSpec((_NB, NREL * NHID), lambda i: (i + 5, 0)),
            pl.BlockSpec((_NB, NREL), lambda i: (i, 0)),
            pl.BlockSpec((_NB, NREL), lambda i: (i, 0)),
            pl.BlockSpec((_NB, NHID), lambda i: (i, 0)),
            pl.BlockSpec((NREL * NHID, OUT), lambda i: (0, 0)),
            pl.BlockSpec((NREL * NHID, OUT), lambda i: (0, 0)),
            pl.BlockSpec((NHID, OUT), lambda i: (0, 0)),
            pl.BlockSpec((NREL, NREL * NHID), lambda i: (0, 0)),
            pl.BlockSpec((OUT, NREL * OUT), lambda i: (0, 0)),
            pl.BlockSpec((1, NREL * OUT), lambda i: (0, 0)),
        ],
        out_specs=[
            pl.BlockSpec((_NB, OUT), lambda i: (i, 0)),
            pl.BlockSpec((_NB, NREL * OUT), lambda i: (i, 0)),
        ],
        out_shape=[
            jax.ShapeDtypeStruct((N, OUT), jnp.float32),
            jax.ShapeDtypeStruct((N, NREL * OUT), jnp.float32),
        ],
    )(U, U, c0, c1, h1, W2a, W2b, W2self, jnp.asarray(_KRON),
      jnp.asarray(_TILE64), embrow)


DSUP = 640
DCK = DSUP // 128
EPW = EPAD // 32
NDSUP = EPW // DSUP
_GROWS = EPAD * 16 // 128


def _dec_gather_body(sp, rp, dp, qt, ht, gp_out,
                     sv, rv, dv, qidx, hidx, qbuf, hbuf, pbuf, gsem):
    c = lax.axis_index("c")
    s = lax.axis_index("s")
    wid = s * 2 + c
    base0 = wid * EPW

    def super_body(u, _):
        base = base0 + u * DSUP
        pltpu.sync_copy(sp.at[pl.ds(base, DSUP)], sv)
        pltpu.sync_copy(rp.at[pl.ds(base, DSUP)], rv)
        pltpu.sync_copy(dp.at[pl.ds(base, DSUP)], dv)
        for j in range(DCK):
            for k in range(8):
                off = j * 128 + k * 16
                svv = sv[pl.ds(off, 16)]
                rvv = rv[pl.ds(off, 16)]
                dvv = dv[pl.ds(off, 16)]
                qidx[j, pl.ds(k * 16, 16)] = svv * NREL + rvv
                hidx[j, pl.ds(k * 16, 16)] = dvv
        descs = [
            pltpu.async_copy(qt.at[qidx.at[j]],
                             qbuf.at[pl.ds(j * 128, 128)], gsem)
            for j in range(DCK)
        ] + [
            pltpu.async_copy(ht.at[hidx.at[j]],
                             hbuf.at[pl.ds(j * 128, 128)], gsem)
            for j in range(DCK)
        ]
        for d in descs:
            d.wait()

        def edge_body(e8, _):
            for k in range(8):
                e = e8 * 8 + k
                acc = ((qbuf[e, pl.ds(0, 16)] * hbuf[e, pl.ds(0, 16)]
                        + qbuf[e, pl.ds(16, 16)] * hbuf[e, pl.ds(16, 16)])
                       + (qbuf[e, pl.ds(32, 16)] * hbuf[e, pl.ds(32, 16)]
                          + qbuf[e, pl.ds(48, 16)] * hbuf[e, pl.ds(48, 16)]))
                pbuf[e8, pl.ds(k * 16, 16)] = acc
            return 0

        lax.fori_loop(0, DSUP // 8, edge_body, 0)
        pltpu.sync_copy(pbuf, gp_out.at[pl.ds(base // 8, DSUP // 8)])
        return 0

    lax.fori_loop(0, NDSUP, super_body, 0)


def _dec_gather(sp, rp, dp, qtab, h2):
    mesh = plsc.VectorSubcoreMesh(core_axis_name="c", subcore_axis_name="s")
    return pl.kernel(
        _dec_gather_body,
        out_type=jax.ShapeDtypeStruct((_GROWS, 128), jnp.float32),
        mesh=mesh,
        scratch_types=[
            pltpu.VMEM((DSUP,), jnp.int32),
            pltpu.VMEM((DSUP,), jnp.int32),
            pltpu.VMEM((DSUP,), jnp.int32),
            pltpu.VMEM((DCK, 128), jnp.int32),
            pltpu.VMEM((DCK, 128), jnp.int32),
            pltpu.VMEM((DSUP, OUT), jnp.float32),
            pltpu.VMEM((DSUP, OUT), jnp.float32),
            pltpu.VMEM((DSUP // 8, 128), jnp.float32),
            pltpu.SemaphoreType.DMA,
        ],
        compiler_params=pltpu.CompilerParams(use_tc_tiling_on_sc=False),
    )(sp, rp, dp, qtab, h2)


_SUM16 = np.kron(np.eye(8, dtype=np.float32), np.ones((16, 1), np.float32))
_GB = 4096


def _dec_body(g_ref, k_ref, o_ref):
    o_ref[...] = jnp.dot(g_ref[...], k_ref[...],
                         preferred_element_type=jnp.float32)


def _decode(gp):
    out = pl.pallas_call(
        _dec_body,
        grid=(_GROWS // _GB,),
        in_specs=[
            pl.BlockSpec((_GB, 128), lambda i: (i, 0)),
            pl.BlockSpec((128, 8), lambda i: (0, 0)),
        ],
        out_specs=pl.BlockSpec((_GB, 8), lambda i: (i, 0)),
        out_shape=jax.ShapeDtypeStruct((_GROWS, 8), jnp.float32),
    )(gp, jnp.asarray(_SUM16))
    return out.reshape(EPAD)[:E]


def kernel(x, W1, W2, rel_emb, src, rel, dst):
    src = src.astype(jnp.int32)
    rel = rel.astype(jnp.int32)
    dst = dst.astype(jnp.int32)

    pad = jnp.zeros((EPAD - E,), jnp.int32)
    sp = jnp.concatenate([src, pad])
    rp = jnp.concatenate([rel, pad])
    dp = jnp.concatenate([dst, pad])

    W1cat = W1[:NP].transpose(1, 0, 2).reshape(NFEAT, NP * NHID)
    XW = _xw1(x, W1cat)
    XWf = XW.reshape(N * NP, NHID)

    U1, cnt0, cnt1 = _make_agg(1, True, NP * N)(sp, rp, dp, XWf)
    U1r = U1.reshape(2 * NPAD, NREL * NHID)
    c0 = cnt0[:N * NREL].reshape(N, NREL)
    c1 = cnt1[:N * NREL].reshape(N, NREL)

    h1 = _h1(U1r, c0, c1, x, W1[NP])

    (U2,) = _make_agg(2, False, N)(sp, rp, dp, h1)
    U2r = U2.reshape(2 * NPAD, NREL * NHID)

    h2, qtab = _h2(U2r, c0, c1, h1,
                   W2[:NREL].reshape(NREL * NHID, OUT),
                   W2[NREL:NP].reshape(NREL * NHID, OUT), W2[NP], rel_emb)

    gp = _dec_gather(sp, rp, dp, qtab.reshape(N * NREL, OUT), h2)
    return _decode(gp)


# speedup vs baseline: 30.0711x; 1.0373x over previous
import functools

import jax
import jax.numpy as jnp
import numpy as np
from jax import lax
from jax.experimental import pallas as pl
from jax.experimental.pallas import tpu as pltpu
from jax.experimental.pallas import tpu_sc as plsc

N = 10000
NREL = 10
E = 320000
NFEAT = 128
NHID = 16
OUT = 64
NP = 2 * NREL
NPAD = 10240
HALF = NREL * NPAD
DUMP = HALF
HROWS = HALF + 64
UCROWS = HALF * NHID // 128

NTILE = 16
EPAD = 327680
EPT = EPAD // NTILE
SUP = 640
NSUP = EPT // SUP
NCK = SUP // 128


def _agg_body(layer, with_counts, *refs):
    if with_counts:
        (sp, rp, dp, table, u_out, cnt_out0, cnt_out1,
         sv, rv, dv, gidx, sidx, rows, r128, ones, zv1, u_sh, cnt_sh,
         gsem) = refs
    else:
        (sp, rp, dp, table, u_out,
         sv, rv, dv, gidx, sidx, rows, r128, u_sh, gsem) = refs

    c = lax.axis_index("c")
    s = lax.axis_index("s")
    zvec = jnp.zeros((16,), jnp.float32)

    def zero_body(i, _):
        rows[i, :] = zvec
        return 0

    lax.fori_loop(0, SUP, zero_body, 0)
    if with_counts:
        def zero1_body(i, _):
            zv1[pl.ds(i * 16, 16)] = zvec
            return 0

        lax.fori_loop(0, 100, zero1_body, 0)
        for k in range(8):
            ones[pl.ds(k * 16, 16)] = jnp.ones((16,), jnp.float32)

    for t in range(10):
        pltpu.sync_copy(rows, u_sh.at[pl.ds(s * 6404 + t * 640, 640)])
    pltpu.sync_copy(rows.at[pl.ds(0, 4)], u_sh.at[pl.ds(s * 6404 + 6400, 4)])
    if with_counts:
        for t in range(4):
            pltpu.sync_copy(zv1, cnt_sh.at[pl.ds(s * 6400 + t * 1600, 1600)])

        @pl.when(s == 0)
        def _():
            pltpu.sync_copy(zv1.at[pl.ds(0, 64)], cnt_sh.at[pl.ds(HALF, 64)])

    plsc.subcore_barrier()

    tile_base = s * EPT
    iota16 = lax.iota(jnp.int32, 16)

    def super_body(u, _):
        base = tile_base + u * SUP
        pltpu.sync_copy(sp.at[pl.ds(base, SUP)], sv)
        pltpu.sync_copy(rp.at[pl.ds(base, SUP)], rv)
        pltpu.sync_copy(dp.at[pl.ds(base, SUP)], dv)
        for j in range(NCK):
            for k in range(8):
                off = j * 128 + k * 16
                svv = sv[pl.ds(off, 16)]
                rvv = rv[pl.ds(off, 16)]
                dvv = dv[pl.ds(off, 16)]
                is0 = c == 0
                if layer == 1:
                    gi = jnp.where(is0, svv * NP + rvv, dvv * NP + rvv + NREL)
                else:
                    gi = jnp.where(is0, svv, dvv)
                si = jnp.where(is0, dvv * NREL + rvv, svv * NREL + rvv)
                pos = base + off + iota16
                si = jnp.where(pos < E, si, DUMP)
                gidx[j, pl.ds(k * 16, 16)] = gi
                sidx[j, pl.ds(k * 16, 16)] = si
        descs = [
            pltpu.async_copy(table.at[gidx.at[j]],
                             rows.at[pl.ds(j * 128, 128)], gsem)
            for j in range(NCK)
        ]
        for d in descs:
            d.wait()
        for j in range(NCK):
            pltpu.sync_copy(rows.at[pl.ds(j * 128, 128)],
                            u_sh.at[sidx.at[j]], add=True)
            if with_counts:
                pltpu.sync_copy(ones, cnt_sh.at[sidx.at[j]], add=True)
        return 0

    lax.fori_loop(0, NSUP, super_body, 0)

    plsc.subcore_barrier()

    def repack_body(i8, _):
        for k in range(8):
            r128[i8, pl.ds(k * 16, 16)] = rows[i8 * 8 + k, :]
        return 0

    for t in range(20):
        pltpu.sync_copy(u_sh.at[pl.ds(s * 6400 + t * 320, 320)],
                        rows.at[pl.ds(0, 320)])
        lax.fori_loop(0, 40, repack_body, 0)
        pltpu.sync_copy(r128,
                        u_out.at[pl.ds(c * UCROWS + s * 800 + t * 40, 40)])

    if with_counts:
        @pl.when(jnp.logical_and(s == 0, c == 0))
        def _():
            pltpu.sync_copy(cnt_sh.at[pl.ds(0, HALF)], cnt_out0)

        @pl.when(jnp.logical_and(s == 0, c == 1))
        def _():
            pltpu.sync_copy(cnt_sh.at[pl.ds(0, HALF)], cnt_out1)


def _make_agg(layer, with_counts, table_rows):
    mesh = plsc.VectorSubcoreMesh(core_axis_name="c", subcore_axis_name="s")
    out_type = [jax.ShapeDtypeStruct((2 * UCROWS, 128), jnp.float32)]
    scratch = [
        pltpu.VMEM((SUP,), jnp.int32),
        pltpu.VMEM((SUP,), jnp.int32),
        pltpu.VMEM((SUP,), jnp.int32),
        pltpu.VMEM((NCK, 128), jnp.int32),
        pltpu.VMEM((NCK, 128), jnp.int32),
        pltpu.VMEM((SUP, NHID), jnp.float32),
        pltpu.VMEM((40, 128), jnp.float32),
    ]
    if with_counts:
        out_type.append(jax.ShapeDtypeStruct((HALF,), jnp.float32))
        out_type.append(jax.ShapeDtypeStruct((HALF,), jnp.float32))
        scratch.append(pltpu.VMEM((128,), jnp.float32))
    if with_counts:
        scratch.append(pltpu.VMEM((1600,), jnp.float32))
    scratch.append(pltpu.VMEM_SHARED((HROWS, NHID), jnp.float32))
    if with_counts:
        scratch.append(pltpu.VMEM_SHARED((HROWS,), jnp.float32))
    scratch.append(pltpu.SemaphoreType.DMA)
    return pl.kernel(
        functools.partial(_agg_body, layer, with_counts),
        out_type=out_type,
        mesh=mesh,
        scratch_types=scratch,
        compiler_params=pltpu.CompilerParams(use_tc_tiling_on_sc=False),
    )


def _xw1_body(x_ref, w_ref, out_ref):
    out_ref[...] = jnp.dot(x_ref[...], w_ref[...],
                           preferred_element_type=jnp.float32)


def _xw1(x, W1cat):
    return pl.pallas_call(
        _xw1_body,
        out_shape=jax.ShapeDtypeStruct((N, NP * NHID), jnp.float32),
    )(x, W1cat)


_KRON = np.kron(np.eye(NREL, dtype=np.float32), np.ones((1, NHID), np.float32))
_SUMT = np.tile(np.eye(NHID, dtype=np.float32), (NREL, 1))


_NB = 2048
_UB = _NB * NREL * NHID // 128


def _h1_body(u0_ref, u1_ref, c0_ref, c1_ref, x_ref, w_ref, kron_ref, sumt_ref,
             out_ref):
    u0 = u0_ref[...]
    u1 = u1_ref[...]
    invb0 = jnp.dot(1.0 / jnp.maximum(c0_ref[...], 1.0), kron_ref[...],
                    preferred_element_type=jnp.float32)
    invb1 = jnp.dot(1.0 / jnp.maximum(c1_ref[...], 1.0), kron_ref[...],
                    preferred_element_type=jnp.float32)
    s = jnp.dot(u0 * invb0 + u1 * invb1, sumt_ref[...],
                preferred_element_type=jnp.float32)
    s = s + jnp.dot(x_ref[...], w_ref[...], preferred_element_type=jnp.float32)
    out_ref[...] = jnp.maximum(s, 0.0)


def _h1(U, c0, c1, x, W1self):
    return pl.pallas_call(
        _h1_body,
        grid=(N // _NB,),
        in_specs=[
            pl.BlockSpec((_NB, NREL * NHID), lambda i: (i, 0)),
            pl.BlockSpec((_NB, NREL * NHID), lambda i: (i + 5, 0)),
            pl.BlockSpec((_NB, NREL), lambda i: (i, 0)),
            pl.BlockSpec((_NB, NREL), lambda i: (i, 0)),
            pl.BlockSpec((_NB, NFEAT), lambda i: (i, 0)),
            pl.BlockSpec((NFEAT, NHID), lambda i: (0, 0)),
            pl.BlockSpec((NREL, NREL * NHID), lambda i: (0, 0)),
            pl.BlockSpec((NREL * NHID, NHID), lambda i: (0, 0)),
        ],
        out_specs=pl.BlockSpec((_NB, NHID), lambda i: (i, 0)),
        out_shape=jax.ShapeDtypeStruct((N, NHID), jnp.float32),
    )(U, U, c0, c1, x, W1self, jnp.asarray(_KRON), jnp.asarray(_SUMT))


_TILE64 = np.tile(np.eye(OUT, dtype=np.float32), (1, NREL))


def _h2_body(u0_ref, u1_ref, c0_ref, c1_ref, h1_ref, w0_ref, w1_ref, ws_ref,
             kron_ref, til_ref, emb_ref, out_ref, q_ref):
    u0 = u0_ref[...]
    u1 = u1_ref[...]
    invb0 = jnp.dot(1.0 / jnp.maximum(c0_ref[...], 1.0), kron_ref[...],
                    preferred_element_type=jnp.float32)
    invb1 = jnp.dot(1.0 / jnp.maximum(c1_ref[...], 1.0), kron_ref[...],
                    preferred_element_type=jnp.float32)
    acc = jnp.dot(h1_ref[...], ws_ref[...], preferred_element_type=jnp.float32)
    acc = acc + jnp.dot(u0 * invb0, w0_ref[...],
                        preferred_element_type=jnp.float32)
    acc = acc + jnp.dot(u1 * invb1, w1_ref[...],
                        preferred_element_type=jnp.float32)
    out_ref[...] = acc
    q_ref[...] = jnp.dot(acc, til_ref[...],
                         preferred_element_type=jnp.float32) * emb_ref[...]


def _h2(U, c0, c1, h1, W2a, W2b, W2self, rel_emb):
    embrow = rel_emb.reshape(1, NREL * OUT)
    return pl.pallas_call(
        _h2_body,
        grid=(N // _NB,),
        in_specs=[
            pl.BlockSpec((_NB, NREL * NHID), lambda i: (i, 0)),
            pl.BlockSpec((_NB, NREL * NHID), lambda i: (i + 5, 0)),
            pl.BlockSpec((_NB, NREL), lambda i: (i, 0)),
            pl.BlockSpec((_NB, NREL), lambda i: (i, 0)),
            pl.BlockSpec((_NB, NHID), lambda i: (i, 0)),
            pl.BlockSpec((NREL * NHID, OUT), lambda i: (0, 0)),
            pl.BlockSpec((NREL * NHID, OUT), lambda i: (0, 0)),
            pl.BlockSpec((NHID, OUT), lambda i: (0, 0)),
            pl.BlockSpec((NREL, NREL * NHID), lambda i: (0, 0)),
            pl.BlockSpec((OUT, NREL * OUT), lambda i: (0, 0)),
            pl.BlockSpec((1, NREL * OUT), lambda i: (0, 0)),
        ],
        out_specs=[
            pl.BlockSpec((_NB, OUT), lambda i: (i, 0)),
            pl.BlockSpec((_NB, NREL * OUT), lambda i: (i, 0)),
        ],
        out_shape=[
            jax.ShapeDtypeStruct((N, OUT), jnp.float32),
            jax.ShapeDtypeStruct((N, NREL * OUT), jnp.float32),
        ],
    )(U, U, c0, c1, h1, W2a, W2b, W2self, jnp.asarray(_KRON),
      jnp.asarray(_TILE64), embrow)


DSUP = 640
DCK = DSUP // 128
EPW = EPAD // 32
NDSUP = EPW // DSUP
_GROWS = EPAD * 16 // 128


_DSPLIT0 = 12800


def _dec_gather_body(sp, rp, dp, qt, ht, gp_out,
                     sv, rv, dv, qidx, hidx, qbuf, hbuf, pbuf, gsem):
    c = lax.axis_index("c")
    s = lax.axis_index("s")
    base0 = jnp.where(c == 0, s * _DSPLIT0,
                      16 * _DSPLIT0 + s * (EPW * 2 - _DSPLIT0))
    nsup = jnp.where(c == 0, _DSPLIT0 // DSUP, (EPW * 2 - _DSPLIT0) // DSUP)

    def super_body(u, _):
        base = base0 + u * DSUP
        pltpu.sync_copy(sp.at[pl.ds(base, DSUP)], sv)
        pltpu.sync_copy(rp.at[pl.ds(base, DSUP)], rv)
        pltpu.sync_copy(dp.at[pl.ds(base, DSUP)], dv)
        for j in range(DCK):
            for k in range(8):
                off = j * 128 + k * 16
                svv = sv[pl.ds(off, 16)]
                rvv = rv[pl.ds(off, 16)]
                dvv = dv[pl.ds(off, 16)]
                qidx[j, pl.ds(k * 16, 16)] = svv * NREL + rvv
                hidx[j, pl.ds(k * 16, 16)] = dvv
        descs = [
            pltpu.async_copy(qt.at[qidx.at[j]],
                             qbuf.at[pl.ds(j * 128, 128)], gsem)
            for j in range(DCK)
        ] + [
            pltpu.async_copy(ht.at[hidx.at[j]],
                             hbuf.at[pl.ds(j * 128, 128)], gsem)
            for j in range(DCK)
        ]
        for d in descs:
            d.wait()

        def edge_body(e8, _):
            for k in range(8):
                e = e8 * 8 + k
                acc = ((qbuf[e, pl.ds(0, 16)] * hbuf[e, pl.ds(0, 16)]
                        + qbuf[e, pl.ds(16, 16)] * hbuf[e, pl.ds(16, 16)])
                       + (qbuf[e, pl.ds(32, 16)] * hbuf[e, pl.ds(32, 16)]
                          + qbuf[e, pl.ds(48, 16)] * hbuf[e, pl.ds(48, 16)]))
                pbuf[e8, pl.ds(k * 16, 16)] = acc
            return 0

        lax.fori_loop(0, DSUP // 8, edge_body, 0)
        pltpu.sync_copy(pbuf, gp_out.at[pl.ds(base // 8, DSUP // 8)])
        return 0

    lax.fori_loop(0, nsup, super_body, 0)


def _dec_gather(sp, rp, dp, qtab, h2):
    mesh = plsc.VectorSubcoreMesh(core_axis_name="c", subcore_axis_name="s")
    return pl.kernel(
        _dec_gather_body,
        out_type=jax.ShapeDtypeStruct((_GROWS, 128), jnp.float32),
        mesh=mesh,
        scratch_types=[
            pltpu.VMEM((DSUP,), jnp.int32),
            pltpu.VMEM((DSUP,), jnp.int32),
            pltpu.VMEM((DSUP,), jnp.int32),
            pltpu.VMEM((DCK, 128), jnp.int32),
            pltpu.VMEM((DCK, 128), jnp.int32),
            pltpu.VMEM((DSUP, OUT), jnp.float32),
            pltpu.VMEM((DSUP, OUT), jnp.float32),
            pltpu.VMEM((DSUP // 8, 128), jnp.float32),
            pltpu.SemaphoreType.DMA,
        ],
        compiler_params=pltpu.CompilerParams(use_tc_tiling_on_sc=False),
    )(sp, rp, dp, qtab, h2)


_SUM16 = np.kron(np.eye(8, dtype=np.float32), np.ones((16, 1), np.float32))
_GB = 4096


def _dec_body(g_ref, k_ref, o_ref):
    o_ref[...] = jnp.dot(g_ref[...], k_ref[...],
                         preferred_element_type=jnp.float32)


def _decode(gp):
    out = pl.pallas_call(
        _dec_body,
        grid=(_GROWS // _GB,),
        in_specs=[
            pl.BlockSpec((_GB, 128), lambda i: (i, 0)),
            pl.BlockSpec((128, 8), lambda i: (0, 0)),
        ],
        out_specs=pl.BlockSpec((_GB, 8), lambda i: (i, 0)),
        out_shape=jax.ShapeDtypeStruct((_GROWS, 8), jnp.float32),
    )(gp, jnp.asarray(_SUM16))
    return out.reshape(EPAD)[:E]


def kernel(x, W1, W2, rel_emb, src, rel, dst):
    src = src.astype(jnp.int32)
    rel = rel.astype(jnp.int32)
    dst = dst.astype(jnp.int32)

    pad = jnp.zeros((EPAD - E,), jnp.int32)
    sp = jnp.concatenate([src, pad])
    rp = jnp.concatenate([rel, pad])
    dp = jnp.concatenate([dst, pad])

    W1cat = W1[:NP].transpose(1, 0, 2).reshape(NFEAT, NP * NHID)
    XW = _xw1(x, W1cat)
    XWf = XW.reshape(N * NP, NHID)

    U1, cnt0, cnt1 = _make_agg(1, True, NP * N)(sp, rp, dp, XWf)
    U1r = U1.reshape(2 * NPAD, NREL * NHID)
    c0 = cnt0[:N * NREL].reshape(N, NREL)
    c1 = cnt1[:N * NREL].reshape(N, NREL)

    h1 = _h1(U1r, c0, c1, x, W1[NP])

    (U2,) = _make_agg(2, False, N)(sp, rp, dp, h1)
    U2r = U2.reshape(2 * NPAD, NREL * NHID)

    h2, qtab = _h2(U2r, c0, c1, h1,
                   W2[:NREL].reshape(NREL * NHID, OUT),
                   W2[NREL:NP].reshape(NREL * NHID, OUT), W2[NP], rel_emb)

    gp = _dec_gather(sp, rp, dp, qtab.reshape(N * NREL, OUT), h2)
    return _decode(gp)
